# baseline (speedup 1.0000x reference)
import functools

import jax
import jax.numpy as jnp
from jax import lax
from jax.experimental import pallas as pl
from jax.experimental.pallas import tpu as pltpu
from jax.experimental.pallas import tpu_sc as plsc

_B = 16384
_D = 16
_NC = 2
_NS = 16
_NW = _NC * _NS
_BPW = _B // _NW
_CHUNK = 128
_NCHUNK = _BPW // _CHUNK
_NBLK = _BPW // _D


def _dot_kernel(users_hbm, items_hbm, u_hbm, v_hbm, out_hbm,
                uidx, vidx, upar, vpar, urows, vrows, outv, sem):
    wid = lax.axis_index("s") * _NC + lax.axis_index("c")
    base = wid * _BPW

    pltpu.sync_copy(users_hbm.at[pl.ds(wid * _NCHUNK, _NCHUNK)], uidx)
    pltpu.sync_copy(items_hbm.at[pl.ds(wid * _NCHUNK, _NCHUNK)], vidx)

    def split(i, _):
        c = i // 8
        o = (i % 8) * 16
        for idxr, parr in ((uidx, upar), (vidx, vpar)):
            x = idxr[c, pl.ds(o, 16)]
            idxr[c, pl.ds(o, 16)] = lax.shift_right_logical(x, 1)
            parr[c, pl.ds(o, 16)] = lax.bitwise_and(x, 1)
        return _

    lax.fori_loop(0, _NCHUNK * 8, split, 0)

    copies = []
    for j in range(_NCHUNK):
        copies.append(pltpu.async_copy(
            u_hbm.at[uidx.at[j]], urows.at[pl.ds(j * _CHUNK, _CHUNK)], sem))
        copies.append(pltpu.async_copy(
            v_hbm.at[vidx.at[j]], vrows.at[pl.ds(j * _CHUNK, _CHUNK)], sem))
    for c in copies:
        c.wait()

    lane = lax.iota(jnp.int32, 16)
    dn = lax.GatherDimensionNumbers(
        offset_dims=(), collapsed_slice_dims=(0,), start_index_map=(0,))

    def perm(a, idx):
        return lax.gather(a, idx[:, None], dn, (1,),
                          mode=lax.GatherScatterMode.PROMISE_IN_BOUNDS)

    def unpack2(packed):
        even = lax.bitcast_convert_type(
            lax.shift_left(packed, jnp.int32(16)), jnp.float32)
        odd = lax.bitcast_convert_type(
            lax.bitwise_and(packed, jnp.int32(-65536)), jnp.float32)
        return even, odd

    sel_msk = [lane == j for j in range(_D)]
    sel_idx = [jnp.full((16,), j, jnp.int32) for j in range(_D)]

    def block(blk, carry):
        r0 = blk * _D
        c = blk // 8
        o = (blk % 8) * 16
        pu_blk = upar[c, pl.ds(o, 16)]
        pv_blk = vpar[c, pl.ds(o, 16)]
        acc = jnp.zeros((16,), jnp.float32)
        for j in range(_D):
            hu = perm(pu_blk, sel_idx[j])
            hv = perm(pv_blk, sel_idx[j])
            align = lane ^ lax.shift_left(hu ^ hv, 3)
            place = lax.shift_left(hu, 3)
            ue, uo = unpack2(urows[r0 + j, :])
            ve, vo = unpack2(vrows[r0 + j, :])
            p = ue * perm(ve, align) + uo * perm(vo, align)
            for d in (4, 2, 1):
                p = p + perm(p, lane ^ d)
            acc = jnp.where(sel_msk[j], perm(p, place), acc)
        outv[pl.ds(r0, _D)] = acc
        return carry

    lax.fori_loop(0, _NBLK, block, 0)

    pltpu.sync_copy(outv, out_hbm.at[pl.ds(base, _BPW)])


@jax.jit
def _hete_mf(users2d, items2d, u16, v16):
    mesh = plsc.VectorSubcoreMesh(core_axis_name="c", subcore_axis_name="s")
    run = functools.partial(
        pl.kernel,
        mesh=mesh,
        compiler_params=pltpu.CompilerParams(
            use_tc_tiling_on_sc=False, needs_layout_passes=False),
        out_type=jax.ShapeDtypeStruct((_B,), jnp.float32),
        scratch_types=[
            pltpu.VMEM((_NCHUNK, _CHUNK), jnp.int32),
            pltpu.VMEM((_NCHUNK, _CHUNK), jnp.int32),
            pltpu.VMEM((_NCHUNK, _CHUNK), jnp.int32),
            pltpu.VMEM((_NCHUNK, _CHUNK), jnp.int32),
            pltpu.VMEM((_BPW, _D), jnp.int32),
            pltpu.VMEM((_BPW, _D), jnp.int32),
            pltpu.VMEM((_BPW,), jnp.float32),
            pltpu.SemaphoreType.DMA,
        ],
    )(_dot_kernel)
    return run(users2d, items2d, u16, v16)


def _pack(T):
    t16 = T[:1000000].astype(jnp.bfloat16).reshape(500000, 16, 2)
    return lax.bitcast_convert_type(t16, jnp.int32)


def kernel(users, items, U, V):
    users2d = users.reshape(_NW * _NCHUNK, _CHUNK)
    items2d = items.reshape(_NW * _NCHUNK, _CHUNK)
    return _hete_mf(users2d, items2d, _pack(U), _pack(V))


# speedup vs baseline: 28.1432x; 28.1432x over previous
import functools

import jax
import jax.numpy as jnp
from jax import lax
from jax.experimental import pallas as pl
from jax.experimental.pallas import tpu as pltpu
from jax.experimental.pallas import tpu_sc as plsc

_B = 16384
_D = 16
_NC = 2
_NS = 16
_NW = _NC * _NS
_BPW = _B // _NW
_CHUNK = 128
_NCHUNK = _BPW // _CHUNK
_NBLK = _BPW // _D


def _dot_kernel(users_hbm, items_hbm, u_hbm, v_hbm, out_hbm,
                uidx, vidx, urows, vrows, outv, sem):
    wid = lax.axis_index("s") * _NC + lax.axis_index("c")
    base = wid * _BPW

    pltpu.sync_copy(users_hbm.at[pl.ds(wid * _NCHUNK, _NCHUNK)], uidx)
    pltpu.sync_copy(items_hbm.at[pl.ds(wid * _NCHUNK, _NCHUNK)], vidx)

    copies = []
    for j in range(_NCHUNK):
        copies.append(pltpu.async_copy(
            u_hbm.at[uidx.at[j]], urows.at[pl.ds(j * _CHUNK, _CHUNK)], sem))
        copies.append(pltpu.async_copy(
            v_hbm.at[vidx.at[j]], vrows.at[pl.ds(j * _CHUNK, _CHUNK)], sem))
    for c in copies:
        c.wait()

    lane = lax.iota(jnp.int32, 16)
    dn = lax.GatherDimensionNumbers(
        offset_dims=(), collapsed_slice_dims=(0,), start_index_map=(0,))

    def perm(a, idx):
        return lax.gather(a, idx[:, None], dn, (1,),
                          mode=lax.GatherScatterMode.PROMISE_IN_BOUNDS)

    def block(blk, carry):
        r0 = blk * _D
        acc = jnp.zeros((16,), jnp.float32)
        for r in range(_D):
            p = urows[r0 + r, :] * vrows[r0 + r, :]
            for d in (8, 4, 2, 1):
                p = p + perm(p, lane ^ d)
            acc = jnp.where(lane == r, p, acc)
        outv[pl.ds(r0, _D)] = acc
        return carry

    lax.fori_loop(0, _NBLK, block, 0)

    pltpu.sync_copy(outv, out_hbm.at[pl.ds(base, _BPW)])


@jax.jit
def _hete_mf(users2d, items2d, U, V):
    mesh = plsc.VectorSubcoreMesh(core_axis_name="c", subcore_axis_name="s")
    run = functools.partial(
        pl.kernel,
        mesh=mesh,
        compiler_params=pltpu.CompilerParams(use_tc_tiling_on_sc=False),
        out_type=jax.ShapeDtypeStruct((_B,), jnp.float32),
        scratch_types=[
            pltpu.VMEM((_NCHUNK, _CHUNK), jnp.int32),
            pltpu.VMEM((_NCHUNK, _CHUNK), jnp.int32),
            pltpu.VMEM((_BPW, _D), jnp.float32),
            pltpu.VMEM((_BPW, _D), jnp.float32),
            pltpu.VMEM((_BPW,), jnp.float32),
            pltpu.SemaphoreType.DMA,
        ],
    )(_dot_kernel)
    return run(users2d, items2d, U, V)


def kernel(users, items, U, V):
    users2d = users.reshape(_NW * _NCHUNK, _CHUNK)
    items2d = items.reshape(_NW * _NCHUNK, _CHUNK)
    return _hete_mf(users2d, items2d, U, V)


# speedup vs baseline: 171.6397x; 6.0988x over previous
import functools

import jax
import jax.numpy as jnp
from jax import lax
from jax.experimental import pallas as pl
from jax.experimental.pallas import tpu as pltpu
from jax.experimental.pallas import tpu_sc as plsc

_B = 16384
_D = 16
_NC = 2
_NS = 16
_NW = _NC * _NS
_BPW = _B // _NW
_NCHUNK = _BPW // 128
_NBLK = _BPW // _D
_WV = 8


def _dot_kernel(users_hbm, items_hbm, ut_hbm, vt_hbm, out_hbm,
                uidx, vidx, bu0, bv0, bu1, bv1, outv, sem0, sem1):
    wid = lax.axis_index("s") * _NC + lax.axis_index("c")
    base = wid * _BPW

    pltpu.sync_copy(users_hbm.at[pl.ds(wid * _NCHUNK, _NCHUNK)], uidx)
    pltpu.sync_copy(items_hbm.at[pl.ds(wid * _NCHUNK, _NCHUNK)], vidx)

    lane = lax.iota(jnp.int32, 16)
    dn = lax.GatherDimensionNumbers(
        offset_dims=(), collapsed_slice_dims=(0,), start_index_map=(0,))

    def perm(a, idx):
        return lax.gather(a, idx[:, None], dn, (1,),
                          mode=lax.GatherScatterMode.PROMISE_IN_BOUNDS)

    def chunk_vecs(c):
        iu = uidx[lax.div(c, 8), pl.ds(lax.rem(c, 8) * 16, 16)]
        iv = vidx[lax.div(c, 8), pl.ds(lax.rem(c, 8) * 16, 16)]
        return iu, iv

    def fire(c, hw, bufu, bufv, sem):
        iu, iv = chunk_vecs(c)
        cu = lax.shift_right_logical(iu, 7)
        cv = lax.shift_right_logical(iv, 7)
        for j in range(_WV):
            ou = pl.multiple_of(cu[hw * _WV + j] * 128, 128)
            ov = pl.multiple_of(cv[hw * _WV + j] * 128, 128)
            pltpu.async_copy(ut_hbm.at[:, pl.ds(ou, 128)],
                             bufu.at[pl.ds(j * _D, _D), :], sem)
            pltpu.async_copy(vt_hbm.at[:, pl.ds(ov, 128)],
                             bufv.at[pl.ds(j * _D, _D), :], sem)

    def drain(bufu, bufv, sem):
        for j in range(_WV):
            pltpu.make_async_copy(ut_hbm.at[:, pl.ds(0, 128)],
                                  bufu.at[pl.ds(j * _D, _D), :], sem).wait()
            pltpu.make_async_copy(vt_hbm.at[:, pl.ds(0, 128)],
                                  bufv.at[pl.ds(j * _D, _D), :], sem).wait()

    def compute(c, hw, bufu, bufv):
        iu, iv = chunk_vecs(c)
        lu = lax.bitwise_and(iu, 127)
        lv = lax.bitwise_and(iv, 127)
        part = jnp.zeros((16,), jnp.float32)
        for j in range(_WV):
            lsu = jnp.full((16,), lu[hw * _WV + j], jnp.int32)
            lsv = jnp.full((16,), lv[hw * _WV + j], jnp.int32)
            ru = plsc.load_gather(bufu, [lane + j * _D, lsu])
            rv = plsc.load_gather(bufv, [lane + j * _D, lsv])
            p = ru * rv
            for d in (8, 4, 2, 1):
                p = p + perm(p, lane ^ d)
            part = jnp.where(lane == hw * _WV + j, p, part)
        return part

    fire(0, 0, bu0, bv0, sem0)

    def body(t, carry):
        fire(t, 1, bu1, bv1, sem1)
        drain(bu0, bv0, sem0)
        acc0 = compute(t, 0, bu0, bv0)

        @pl.when(t < _NBLK - 1)
        def _():
            fire(t + 1, 0, bu0, bv0, sem0)

        drain(bu1, bv1, sem1)
        acc1 = compute(t, 1, bu1, bv1)
        outv[pl.ds(t * _D, _D)] = acc0 + acc1
        return carry

    lax.fori_loop(0, _NBLK, body, 0)

    pltpu.sync_copy(outv, out_hbm.at[pl.ds(base, _BPW)])


@jax.jit
def _hete_mf(users2d, items2d, ut, vt):
    mesh = plsc.VectorSubcoreMesh(core_axis_name="c", subcore_axis_name="s")
    run = functools.partial(
        pl.kernel,
        mesh=mesh,
        compiler_params=pltpu.CompilerParams(
            use_tc_tiling_on_sc=True, needs_layout_passes=False),
        out_type=jax.ShapeDtypeStruct((_B,), jnp.float32),
        scratch_types=[
            pltpu.VMEM((_NCHUNK, 128), jnp.int32),
            pltpu.VMEM((_NCHUNK, 128), jnp.int32),
            pltpu.VMEM((_WV * _D, 128), jnp.float32),
            pltpu.VMEM((_WV * _D, 128), jnp.float32),
            pltpu.VMEM((_WV * _D, 128), jnp.float32),
            pltpu.VMEM((_WV * _D, 128), jnp.float32),
            pltpu.VMEM((_BPW,), jnp.float32),
            pltpu.SemaphoreType.DMA,
            pltpu.SemaphoreType.DMA,
        ],
    )(_dot_kernel)
    return run(users2d, items2d, ut, vt)


def kernel(users, items, U, V):
    users2d = users.reshape(_NW * _NCHUNK, 128)
    items2d = items.reshape(_NW * _NCHUNK, 128)
    return _hete_mf(users2d, items2d, U.T, V.T)
